# Initial kernel scaffold; baseline (speedup 1.0000x reference)
import jax
import jax.numpy as jnp
from jax.experimental import pallas as pl


def kernel(x_paper, x_author, ei_awp, ei_pwa, ei_pcp, Win, b_in, KW, Kb, QW, Qb, VW, Vb, AW, Ab, skip, RW_att, RW_msg, prel, Wout, bout):
    raise NotImplementedError("write your pallas kernel here")


# speedup vs baseline: 6.5341x; 6.5341x over previous
import functools
import math

import jax
import jax.numpy as jnp
from jax.experimental import pallas as pl

_NT = 2
_L = 2
_H = 4
_D = 128
_DH = _D // _H
_N = 50000
_E = 200000
_EDGE_META = [(1, 0), (0, 1), (0, 0)]

_N_TILE = 1000
_E_TILE = 2000


def _mm_kernel(x_ref, w_ref, b_ref, o_ref, *, act):
    y = jnp.dot(x_ref[...], w_ref[...], preferred_element_type=jnp.float32)
    y = y + b_ref[...]
    if act == "relu":
        y = jnp.maximum(y, 0.0)
    o_ref[...] = y


def _mm(x, W, b, act="none"):
    n = x.shape[0]
    return pl.pallas_call(
        functools.partial(_mm_kernel, act=act),
        grid=(n // _N_TILE,),
        in_specs=[
            pl.BlockSpec((_N_TILE, _D), lambda i: (i, 0)),
            pl.BlockSpec((_D, _D), lambda i: (0, 0)),
            pl.BlockSpec((1, _D), lambda i: (0, 0)),
        ],
        out_specs=pl.BlockSpec((_N_TILE, _D), lambda i: (i, 0)),
        out_shape=jax.ShapeDtypeStruct((n, _D), jnp.float32),
    )(x, W, b.reshape(1, _D))


def _edge_kernel(k_ref, q_ref, v_ref, ba_ref, bm_ref, s_ref, p_ref,
                 sc_ref, ve_ref):
    ke = jnp.dot(k_ref[...], ba_ref[...], preferred_element_type=jnp.float32)
    ve = jnp.dot(v_ref[...], bm_ref[...], preferred_element_type=jnp.float32)
    sc = jnp.dot(ke * q_ref[...], s_ref[...],
                 preferred_element_type=jnp.float32)
    sc_ref[...] = sc * p_ref[...]
    ve_ref[...] = ve


def _edge_compute(k_src, q_dst, v_src, ba, bm, smat, pscale):
    return pl.pallas_call(
        _edge_kernel,
        grid=(_E // _E_TILE,),
        in_specs=[
            pl.BlockSpec((_E_TILE, _D), lambda i: (i, 0)),
            pl.BlockSpec((_E_TILE, _D), lambda i: (i, 0)),
            pl.BlockSpec((_E_TILE, _D), lambda i: (i, 0)),
            pl.BlockSpec((_D, _D), lambda i: (0, 0)),
            pl.BlockSpec((_D, _D), lambda i: (0, 0)),
            pl.BlockSpec((_D, _H), lambda i: (0, 0)),
            pl.BlockSpec((1, _H), lambda i: (0, 0)),
        ],
        out_specs=(
            pl.BlockSpec((_E_TILE, _H), lambda i: (i, 0)),
            pl.BlockSpec((_E_TILE, _D), lambda i: (i, 0)),
        ),
        out_shape=(
            jax.ShapeDtypeStruct((_E, _H), jnp.float32),
            jax.ShapeDtypeStruct((_E, _D), jnp.float32),
        ),
    )(k_src, q_dst, v_src, ba, bm, smat, pscale)


def _agg_kernel(a_ref, h_ref, w_ref, b_ref, al_ref, o_ref):
    g = jax.nn.gelu(a_ref[...])
    y = jnp.dot(g, w_ref[...], preferred_element_type=jnp.float32)
    y = y + b_ref[...]
    al = al_ref[...]
    o = al * y + (1.0 - al) * h_ref[...]
    o_ref[...] = jnp.maximum(o, 0.0)


def _agg_update(agg, h_prev, W, b, alpha):
    return pl.pallas_call(
        _agg_kernel,
        grid=(_N // _N_TILE,),
        in_specs=[
            pl.BlockSpec((_N_TILE, _D), lambda i: (i, 0)),
            pl.BlockSpec((_N_TILE, _D), lambda i: (i, 0)),
            pl.BlockSpec((_D, _D), lambda i: (0, 0)),
            pl.BlockSpec((1, _D), lambda i: (0, 0)),
            pl.BlockSpec((1, 1), lambda i: (0, 0)),
        ],
        out_specs=pl.BlockSpec((_N_TILE, _D), lambda i: (i, 0)),
        out_shape=jax.ShapeDtypeStruct((_N, _D), jnp.float32),
    )(agg, h_prev, W, b.reshape(1, _D), alpha.reshape(1, 1))


def _block_diag(mats):
    out = jnp.zeros((_D, _D), dtype=mats.dtype)
    for i in range(_H):
        out = out.at[i * _DH:(i + 1) * _DH, i * _DH:(i + 1) * _DH].set(mats[i])
    return out


def kernel(x_paper, x_author, ei_awp, ei_pwa, ei_pcp, Win, b_in, KW, Kb,
           QW, Qb, VW, Vb, AW, Ab, skip, RW_att, RW_msg, prel, Wout, bout):
    smat = jnp.kron(jnp.eye(_H, dtype=jnp.float32),
                    jnp.ones((_DH, 1), dtype=jnp.float32))
    edges = [ei_awp, ei_pwa, ei_pcp]

    h = [
        _mm(x_paper, Win[0], b_in[0], act="relu"),
        _mm(x_author, Win[1], b_in[1], act="relu"),
    ]

    for l in range(_L):
        k = [_mm(h[t], KW[l, t], Kb[l, t]) for t in range(_NT)]
        q = [_mm(h[t], QW[l, t], Qb[l, t]) for t in range(_NT)]
        v = [_mm(h[t], VW[l, t], Vb[l, t]) for t in range(_NT)]

        sc_d = [[] for _ in range(_NT)]
        ve_d = [[] for _ in range(_NT)]
        dd_d = [[] for _ in range(_NT)]
        for r, (st, dt) in enumerate(_EDGE_META):
            src, dst = edges[r][0], edges[r][1]
            k_src = jnp.take(k[st], src, axis=0)
            q_dst = jnp.take(q[dt], dst, axis=0)
            v_src = jnp.take(v[st], src, axis=0)
            ba = _block_diag(RW_att[l, r])
            bm = _block_diag(RW_msg[l, r])
            pscale = (prel[l, r] / math.sqrt(_DH)).reshape(1, _H)
            sc, ve = _edge_compute(k_src, q_dst, v_src, ba, bm, smat, pscale)
            sc_d[dt].append(sc)
            ve_d[dt].append(ve)
            dd_d[dt].append(dst)

        h_new = []
        for t in range(_NT):
            sc = jnp.concatenate(sc_d[t], 0)
            ve = jnp.concatenate(ve_d[t], 0)
            dd = jnp.concatenate(dd_d[t], 0)
            m = jax.ops.segment_max(sc, dd, num_segments=_N)
            m = jnp.where(jnp.isfinite(m), m, 0.0)
            e = jnp.exp(sc - m[dd])
            s = jax.ops.segment_sum(e, dd, num_segments=_N)
            att = e / (s[dd] + 1e-16)
            att_exp = jnp.repeat(att, _DH, axis=1)
            agg = jax.ops.segment_sum(ve * att_exp, dd, num_segments=_N)
            alpha = jax.nn.sigmoid(skip[l, t])
            h_new.append(_agg_update(agg, h[t], AW[l, t], Ab[l, t], alpha))
        h = h_new

    return (
        _mm(h[0], Wout[0], bout[0]),
        _mm(h[1], Wout[1], bout[1]),
    )


# speedup vs baseline: 7.0229x; 1.0748x over previous
import functools
import math

import jax
import jax.numpy as jnp
from jax.experimental import pallas as pl

_NT = 2
_L = 2
_H = 4
_D = 128
_DH = _D // _H
_N = 50000
_E = 200000
_EDGE_META = [(1, 0), (0, 1), (0, 0)]

_N_TILE = 1000
_E_TILE = 2000


def _mm_kernel(x_ref, w_ref, b_ref, o_ref, *, act):
    y = jnp.dot(x_ref[...], w_ref[...], preferred_element_type=jnp.float32)
    y = y + b_ref[...]
    if act == "relu":
        y = jnp.maximum(y, 0.0)
    o_ref[...] = y


def _mm(x, W, b, act="none"):
    n = x.shape[0]
    dout = W.shape[1]
    return pl.pallas_call(
        functools.partial(_mm_kernel, act=act),
        grid=(n // _N_TILE,),
        in_specs=[
            pl.BlockSpec((_N_TILE, _D), lambda i: (i, 0)),
            pl.BlockSpec((_D, dout), lambda i: (0, 0)),
            pl.BlockSpec((1, dout), lambda i: (0, 0)),
        ],
        out_specs=pl.BlockSpec((_N_TILE, dout), lambda i: (i, 0)),
        out_shape=jax.ShapeDtypeStruct((n, dout), jnp.float32),
    )(x, W, b.reshape(1, dout))


def _edge_kernel(kv_ref, q_ref, ba_ref, bm_ref, s_ref, p_ref,
                 sc_ref, ve_ref):
    k = kv_ref[:, :_D]
    v = kv_ref[:, _D:]
    ke = jnp.dot(k, ba_ref[...], preferred_element_type=jnp.float32)
    ve = jnp.dot(v, bm_ref[...], preferred_element_type=jnp.float32)
    sc = jnp.dot(ke * q_ref[...], s_ref[...],
                 preferred_element_type=jnp.float32)
    sc_ref[...] = sc * p_ref[...]
    ve_ref[...] = ve


def _edge_compute(kv_src, q_dst, ba, bm, smat, pscale):
    return pl.pallas_call(
        _edge_kernel,
        grid=(_E // _E_TILE,),
        in_specs=[
            pl.BlockSpec((_E_TILE, 2 * _D), lambda i: (i, 0)),
            pl.BlockSpec((_E_TILE, _D), lambda i: (i, 0)),
            pl.BlockSpec((_D, _D), lambda i: (0, 0)),
            pl.BlockSpec((_D, _D), lambda i: (0, 0)),
            pl.BlockSpec((_D, _H), lambda i: (0, 0)),
            pl.BlockSpec((1, _H), lambda i: (0, 0)),
        ],
        out_specs=(
            pl.BlockSpec((_E_TILE, _H), lambda i: (i, 0)),
            pl.BlockSpec((_E_TILE, _D), lambda i: (i, 0)),
        ),
        out_shape=(
            jax.ShapeDtypeStruct((_E, _H), jnp.float32),
            jax.ShapeDtypeStruct((_E, _D), jnp.float32),
        ),
    )(kv_src, q_dst, ba, bm, smat, pscale)


def _agg_kernel(a_ref, h_ref, w_ref, b_ref, al_ref, o_ref):
    g = jax.nn.gelu(a_ref[...])
    y = jnp.dot(g, w_ref[...], preferred_element_type=jnp.float32)
    y = y + b_ref[...]
    al = al_ref[...]
    o = al * y + (1.0 - al) * h_ref[...]
    o_ref[...] = jnp.maximum(o, 0.0)


def _agg_update(agg, h_prev, W, b, alpha):
    return pl.pallas_call(
        _agg_kernel,
        grid=(_N // _N_TILE,),
        in_specs=[
            pl.BlockSpec((_N_TILE, _D), lambda i: (i, 0)),
            pl.BlockSpec((_N_TILE, _D), lambda i: (i, 0)),
            pl.BlockSpec((_D, _D), lambda i: (0, 0)),
            pl.BlockSpec((1, _D), lambda i: (0, 0)),
            pl.BlockSpec((1, 1), lambda i: (0, 0)),
        ],
        out_specs=pl.BlockSpec((_N_TILE, _D), lambda i: (i, 0)),
        out_shape=jax.ShapeDtypeStruct((_N, _D), jnp.float32),
    )(agg, h_prev, W, b.reshape(1, _D), alpha.reshape(1, 1))


def _block_diag(mats):
    out = jnp.zeros((_D, _D), dtype=mats.dtype)
    for i in range(_H):
        out = out.at[i * _DH:(i + 1) * _DH, i * _DH:(i + 1) * _DH].set(mats[i])
    return out


def kernel(x_paper, x_author, ei_awp, ei_pwa, ei_pcp, Win, b_in, KW, Kb,
           QW, Qb, VW, Vb, AW, Ab, skip, RW_att, RW_msg, prel, Wout, bout):
    smat = jnp.kron(jnp.eye(_H, dtype=jnp.float32),
                    jnp.ones((_DH, 1), dtype=jnp.float32))
    edges = [ei_awp, ei_pwa, ei_pcp]

    h = [
        _mm(x_paper, Win[0], b_in[0], act="relu"),
        _mm(x_author, Win[1], b_in[1], act="relu"),
    ]

    for l in range(_L):
        kv = []
        q = []
        for t in range(_NT):
            Wkvq = jnp.concatenate([KW[l, t], VW[l, t], QW[l, t]], axis=1)
            bkvq = jnp.concatenate([Kb[l, t], Vb[l, t], Qb[l, t]], axis=0)
            kvq = _mm(h[t], Wkvq, bkvq)
            kv.append(kvq[:, :2 * _D])
            q.append(kvq[:, 2 * _D:])

        sc_d = [[] for _ in range(_NT)]
        ve_d = [[] for _ in range(_NT)]
        dd_d = [[] for _ in range(_NT)]
        for r, (st, dt) in enumerate(_EDGE_META):
            src, dst = edges[r][0], edges[r][1]
            kv_src = jnp.take(kv[st], src, axis=0)
            q_dst = jnp.take(q[dt], dst, axis=0)
            ba = _block_diag(RW_att[l, r])
            bm = _block_diag(RW_msg[l, r])
            pscale = (prel[l, r] / math.sqrt(_DH)).reshape(1, _H)
            sc, ve = _edge_compute(kv_src, q_dst, ba, bm, smat, pscale)
            sc_d[dt].append(sc)
            ve_d[dt].append(ve)
            dd_d[dt].append(dst)

        h_new = []
        for t in range(_NT):
            sc = jnp.concatenate(sc_d[t], 0)
            ve = jnp.concatenate(ve_d[t], 0)
            dd = jnp.concatenate(dd_d[t], 0)
            m = jax.ops.segment_max(sc, dd, num_segments=_N)
            m = jnp.where(jnp.isfinite(m), m, 0.0)
            e = jnp.exp(sc - m[dd])
            s = jax.ops.segment_sum(e, dd, num_segments=_N)
            att = e / (s[dd] + 1e-16)
            att_exp = jnp.repeat(att, _DH, axis=1)
            agg = jax.ops.segment_sum(ve * att_exp, dd, num_segments=_N)
            alpha = jax.nn.sigmoid(skip[l, t])
            h_new.append(_agg_update(agg, h[t], AW[l, t], Ab[l, t], alpha))
        h = h_new

    return (
        _mm(h[0], Wout[0], bout[0]),
        _mm(h[1], Wout[1], bout[1]),
    )
